# baseline (speedup 1.0000x reference)
import functools

import jax
import jax.numpy as jnp
from jax import lax
from jax.experimental import pallas as pl
from jax.experimental.pallas import tpu as pltpu
from jax.experimental.pallas import tpu_sc as plsc

NODES = 50000
NP = 50048
NUSERS = 25000
D = 64
DH = 32
E = 800000
CH = 128
CHUNKS = 6336
EPAD = CHUNKS * CH
TILES = 16
CPT = CHUNKS // TILES
K = 6
BLOCKS = CPT // K
ROWS_PT = NP // TILES
WB = 136
NWB = ROWS_PT // WB

_mesh = plsc.VectorSubcoreMesh(core_axis_name="c", subcore_axis_name="s")


@functools.partial(
    pl.kernel,
    mesh=_mesh,
    compiler_params=pltpu.CompilerParams(use_tc_tiling_on_sc=False),
    out_type=(
        jax.ShapeDtypeStruct((2 * NP, DH), jnp.float32),
        jax.ShapeDtypeStruct((2 * NP, DH), jnp.float32),
    ),
    scratch_types=[
        pltpu.VMEM((K, CH), jnp.int32),
        pltpu.VMEM((K, CH), jnp.int32),
        pltpu.VMEM((K, CH, DH), jnp.float32),
        pltpu.VMEM((WB, DH), jnp.float32),
        pltpu.VMEM_SHARED((NP, DH), jnp.float32),
        pltpu.SemaphoreType.DMA,
        pltpu.SemaphoreType.DMA,
    ],
)
def _propagate(tab0, rows_hbm, cols_hbm, zeros_hbm, x1_out, x2_out,
               rows_v, cols_v, gbuf, zbuf, acc, gsem, zsem):
    c = lax.axis_index("c")
    s = lax.axis_index("s")
    coff = c * NP

    def zero_acc():
        pltpu.sync_copy(zeros_hbm, zbuf)
        cps = []
        for k in range(NWB):
            cps.append(pltpu.async_copy(
                zbuf, acc.at[pl.ds(s * ROWS_PT + k * WB, WB)], zsem))
        for cp in cps:
            cp.wait()

    def layer(src_tab, dst_tab):
        zero_acc()
        plsc.subcore_barrier()

        base = s * CPT

        def body(b, carry):
            chunk0 = base + b * K
            pltpu.sync_copy(rows_hbm.at[pl.ds(chunk0, K)], rows_v)
            pltpu.sync_copy(cols_hbm.at[c, pl.ds(chunk0, K)], cols_v)
            cps = [
                pltpu.async_copy(src_tab.at[cols_v.at[j]], gbuf.at[j], gsem)
                for j in range(K)
            ]
            for cp in cps:
                cp.wait()
            for j in range(K):
                pltpu.sync_copy(gbuf.at[j], acc.at[rows_v.at[j]], add=True)
            return carry

        lax.fori_loop(0, BLOCKS, body, 0)
        plsc.subcore_barrier()

        for k in range(NWB):
            r0 = s * ROWS_PT + k * WB
            pltpu.sync_copy(acc.at[pl.ds(r0, WB)], zbuf)
            pltpu.sync_copy(zbuf, dst_tab.at[pl.ds(coff + r0, WB)])
        plsc.subcore_barrier()

    layer(tab0, x1_out)
    layer(x1_out, x2_out)


def _mean_body(x0a, x1a, x2a, x0b, x1b, x2b, o_ref):
    left = x0a[...] + x1a[...] + x2a[...]
    right = x0b[...] + x1b[...] + x2b[...]
    o_ref[...] = jnp.concatenate([left, right], axis=1) * (1.0 / 3.0)


_MEAN_BLOCK = 3128


def _mean3(x0, x1, x2):
    grid = NP // _MEAN_BLOCK
    sa = pl.BlockSpec((_MEAN_BLOCK, DH), lambda i: (i, 0))
    sb = pl.BlockSpec((_MEAN_BLOCK, DH), lambda i: (i + NP // _MEAN_BLOCK, 0))
    return pl.pallas_call(
        _mean_body,
        grid=(grid,),
        in_specs=[sa, sa, sa, sb, sb, sb],
        out_specs=pl.BlockSpec((_MEAN_BLOCK, D), lambda i: (i, 0)),
        out_shape=jax.ShapeDtypeStruct((NP, D), jnp.float32),
    )(x0, x1, x2, x0, x1, x2)


def kernel(edge_index, emb_weight):
    row = edge_index[0]
    col = edge_index[1]
    pad = EPAD - E
    rows = jnp.concatenate(
        [row, jnp.full((pad,), NODES, jnp.int32)]).reshape(CHUNKS, CH)
    colp = jnp.concatenate([col, jnp.zeros((pad,), jnp.int32)])
    cols = jnp.stack([colp, colp + NP]).reshape(2, CHUNKS, CH)
    embp = jnp.pad(emb_weight, ((0, NP - NODES), (0, 0)))
    tab0 = jnp.concatenate([embp[:, :DH], embp[:, DH:]], axis=0)
    zeros = jnp.zeros((WB, DH), jnp.float32)

    x1_tab, x2_tab = _propagate(tab0, rows, cols, zeros)

    z = _mean3(tab0, x1_tab, x2_tab)
    return z[:NUSERS], z[NUSERS:NODES]


# speedup vs baseline: 1.7039x; 1.7039x over previous
import functools

import jax
import jax.numpy as jnp
from jax import lax
from jax.experimental import pallas as pl
from jax.experimental.pallas import tpu as pltpu
from jax.experimental.pallas import tpu_sc as plsc

NODES = 50000
NP = 50048
NUSERS = 25000
D = 64
DH = 32
E = 800000
CH = 128
CHUNKS = 6272
EPAD = CHUNKS * CH
TILES = 16
CPT = CHUNKS // TILES
K = 4
SBK = 28
SBLOCKS = CPT // SBK
KB = SBK // K
ROWS_PT = NP // TILES
WB = 136
NWB = ROWS_PT // WB

_mesh = plsc.VectorSubcoreMesh(core_axis_name="c", subcore_axis_name="s")


@functools.partial(
    pl.kernel,
    mesh=_mesh,
    compiler_params=pltpu.CompilerParams(use_tc_tiling_on_sc=False),
    out_type=(
        jax.ShapeDtypeStruct((2 * NP, DH), jnp.float32),
        jax.ShapeDtypeStruct((2 * NP, DH), jnp.float32),
    ),
    scratch_types=[
        pltpu.VMEM((SBK, CH), jnp.int32),
        pltpu.VMEM((SBK, CH), jnp.int32),
        pltpu.VMEM((K, CH, DH), jnp.float32),
        pltpu.VMEM((WB, DH), jnp.float32),
        pltpu.VMEM_SHARED((NP, DH), jnp.float32),
        pltpu.SemaphoreType.DMA,
        pltpu.SemaphoreType.DMA,
        pltpu.SemaphoreType.DMA,
    ],
)
def _propagate(tab0, rows_hbm, cols_hbm, zeros_hbm, x1_out, x2_out,
               rows_v, cols_v, gbuf, zbuf, acc, gsem, ssem, zsem):
    c = lax.axis_index("c")
    s = lax.axis_index("s")
    coff = c * NP

    def zero_acc():
        pltpu.sync_copy(zeros_hbm, zbuf)
        cps = []
        for k in range(NWB):
            cps.append(pltpu.async_copy(
                zbuf, acc.at[pl.ds(s * ROWS_PT + k * WB, WB)], zsem))
        for cp in cps:
            cp.wait()

    def layer(src_tab, dst_tab):
        zero_acc()
        plsc.subcore_barrier()

        base = s * CPT

        def sbody(bb, carry):
            chunk0 = base + bb * SBK
            i1 = pltpu.async_copy(rows_hbm.at[pl.ds(chunk0, SBK)],
                                  rows_v, zsem)
            i2 = pltpu.async_copy(cols_hbm.at[c, pl.ds(chunk0, SBK)],
                                  cols_v, zsem)
            i1.wait()
            i2.wait()
            for sb in range(KB):
                j0 = sb * K
                gcps = [
                    pltpu.async_copy(src_tab.at[cols_v.at[j0 + j]],
                                     gbuf.at[j], gsem)
                    for j in range(K)
                ]
                scps = []
                for j in range(K):
                    gcps[j].wait()
                    scps.append(pltpu.async_copy(
                        gbuf.at[j], acc.at[rows_v.at[j0 + j]], ssem,
                        add=True))
                for cp in scps:
                    cp.wait()
            return carry

        lax.fori_loop(0, SBLOCKS, sbody, 0)
        plsc.subcore_barrier()

        for k in range(NWB):
            r0 = s * ROWS_PT + k * WB
            pltpu.sync_copy(acc.at[pl.ds(r0, WB)], zbuf)
            pltpu.sync_copy(zbuf, dst_tab.at[pl.ds(coff + r0, WB)])
        plsc.subcore_barrier()

    layer(tab0, x1_out)
    layer(x1_out, x2_out)


def _mean_body(x0_ref, x1_ref, x2_ref, o_ref):
    o_ref[...] = (x0_ref[...] + x1_ref[...] + x2_ref[...]) * (1.0 / 3.0)


_MEAN_BLOCK = 3128


def _mean3(x0, x1, x2):
    n = x0.shape[0]
    grid = n // _MEAN_BLOCK
    spec = pl.BlockSpec((_MEAN_BLOCK, 128), lambda i: (i, 0))
    return pl.pallas_call(
        _mean_body,
        grid=(grid,),
        in_specs=[spec, spec, spec],
        out_specs=spec,
        out_shape=jax.ShapeDtypeStruct((n, 128), jnp.float32),
    )(x0, x1, x2)


def kernel(edge_index, emb_weight):
    row = edge_index[0]
    col = edge_index[1]
    pad = EPAD - E
    rows = jnp.concatenate(
        [row, jnp.full((pad,), NODES, jnp.int32)]).reshape(CHUNKS, CH)
    colp = jnp.concatenate([col, jnp.zeros((pad,), jnp.int32)])
    cols = jnp.stack([colp, colp + NP]).reshape(2, CHUNKS, CH)
    embp = jnp.pad(emb_weight, ((0, NP - NODES), (0, 0)))
    tab0 = jnp.concatenate([embp[:, :DH], embp[:, DH:]], axis=0)
    zeros = jnp.zeros((WB, DH), jnp.float32)

    x1_tab, x2_tab = _propagate(tab0, rows, cols, zeros)

    zt = _mean3(tab0.reshape(2 * NP * DH // 128, 128),
                x1_tab.reshape(2 * NP * DH // 128, 128),
                x2_tab.reshape(2 * NP * DH // 128, 128)).reshape(2 * NP, DH)
    z = jnp.concatenate([zt[:NODES], zt[NP:NP + NODES]], axis=1)
    return z[:NUSERS], z[NUSERS:NODES]


# speedup vs baseline: 1.7809x; 1.0452x over previous
import functools

import jax
import jax.numpy as jnp
from jax import lax
from jax.experimental import pallas as pl
from jax.experimental.pallas import tpu as pltpu
from jax.experimental.pallas import tpu_sc as plsc

NODES = 50000
NP = 50048
NUSERS = 25000
D = 64
DH = 32
E = 800000
CH = 128
CHUNKS = 6272
EPAD = CHUNKS * CH
TILES = 16
CPT = CHUNKS // TILES
K = 2
SBK = 28
SBLOCKS = CPT // SBK
KB = SBK // K
ROWS_PT = NP // TILES
WB = 136
NWB = ROWS_PT // WB

_mesh = plsc.VectorSubcoreMesh(core_axis_name="c", subcore_axis_name="s")


@functools.partial(
    pl.kernel,
    mesh=_mesh,
    compiler_params=pltpu.CompilerParams(use_tc_tiling_on_sc=False),
    out_type=(
        jax.ShapeDtypeStruct((2 * NP, DH), jnp.float32),
        jax.ShapeDtypeStruct((2 * NP, DH), jnp.float32),
    ),
    scratch_types=[
        pltpu.VMEM((SBK, CH), jnp.int32),
        pltpu.VMEM((SBK, CH), jnp.int32),
        pltpu.VMEM((2, K, CH, DH), jnp.float32),
        pltpu.VMEM((WB, DH), jnp.float32),
        pltpu.VMEM_SHARED((NP, DH), jnp.float32),
        pltpu.SemaphoreType.DMA,
        pltpu.SemaphoreType.DMA,
        pltpu.SemaphoreType.DMA,
        pltpu.SemaphoreType.DMA,
        pltpu.SemaphoreType.DMA,
    ],
)
def _propagate(tab0, rows_hbm, cols_hbm, zeros_hbm, x1_out, x2_out,
               rows_v, cols_v, gbuf, zbuf, acc,
               gsem0, gsem1, ssem0, ssem1, zsem):
    c = lax.axis_index("c")
    s = lax.axis_index("s")
    coff = c * NP
    gsem = (gsem0, gsem1)
    ssem = (ssem0, ssem1)

    def zero_acc():
        pltpu.sync_copy(zeros_hbm, zbuf)
        cps = []
        for k in range(NWB):
            cps.append(pltpu.async_copy(
                zbuf, acc.at[pl.ds(s * ROWS_PT + k * WB, WB)], zsem))
        for cp in cps:
            cp.wait()

    def layer(src_tab, dst_tab):
        zero_acc()
        plsc.subcore_barrier()

        base = s * CPT

        def fire_gathers(g, j0):
            return [
                pltpu.async_copy(src_tab.at[cols_v.at[j0 + j]],
                                 gbuf.at[g, j], gsem[g])
                for j in range(K)
            ]

        def fire_scatters(g, j0):
            return [
                pltpu.async_copy(gbuf.at[g, j], acc.at[rows_v.at[j0 + j]],
                                 ssem[g], add=True)
                for j in range(K)
            ]

        def drain_scatters(g):
            for j in range(K):
                pltpu.make_async_copy(gbuf.at[g, j], acc.at[rows_v.at[j]],
                                      ssem[g]).wait()

        def sbody(bb, carry):
            @pl.when(bb > 0)
            def _():
                drain_scatters(0)
                drain_scatters(1)
            chunk0 = base + bb * SBK
            i1 = pltpu.async_copy(rows_hbm.at[pl.ds(chunk0, SBK)],
                                  rows_v, zsem)
            i2 = pltpu.async_copy(cols_hbm.at[c, pl.ds(chunk0, SBK)],
                                  cols_v, zsem)
            i1.wait()
            i2.wait()
            gcps = {0: fire_gathers(0, 0)}
            scps = {}
            for k in range(KB):
                g = k % 2
                o = 1 - g
                if k + 1 < KB:
                    if k >= 1:
                        for cp in scps[k - 1]:
                            cp.wait()
                    gcps[o] = fire_gathers(o, (k + 1) * K)
                for cp in gcps[g]:
                    cp.wait()
                scps[k] = fire_scatters(g, k * K)
            return carry

        lax.fori_loop(0, SBLOCKS, sbody, 0)
        drain_scatters(0)
        drain_scatters(1)
        plsc.subcore_barrier()

        for k in range(NWB):
            r0 = s * ROWS_PT + k * WB
            pltpu.sync_copy(acc.at[pl.ds(r0, WB)], zbuf)
            pltpu.sync_copy(zbuf, dst_tab.at[pl.ds(coff + r0, WB)])
        plsc.subcore_barrier()

    layer(tab0, x1_out)
    layer(x1_out, x2_out)


def _mean_body(x0_ref, x1_ref, x2_ref, o_ref):
    o_ref[...] = (x0_ref[...] + x1_ref[...] + x2_ref[...]) * (1.0 / 3.0)


_MEAN_BLOCK = 3128


def _mean3(x0, x1, x2):
    n = x0.shape[0]
    grid = n // _MEAN_BLOCK
    spec = pl.BlockSpec((_MEAN_BLOCK, 128), lambda i: (i, 0))
    return pl.pallas_call(
        _mean_body,
        grid=(grid,),
        in_specs=[spec, spec, spec],
        out_specs=spec,
        out_shape=jax.ShapeDtypeStruct((n, 128), jnp.float32),
    )(x0, x1, x2)


def kernel(edge_index, emb_weight):
    row = edge_index[0]
    col = edge_index[1]
    pad = EPAD - E
    rows = jnp.concatenate(
        [row, jnp.full((pad,), NODES, jnp.int32)]).reshape(CHUNKS, CH)
    colp = jnp.concatenate([col, jnp.zeros((pad,), jnp.int32)])
    cols = jnp.stack([colp, colp + NP]).reshape(2, CHUNKS, CH)
    embp = jnp.pad(emb_weight, ((0, NP - NODES), (0, 0)))
    tab0 = jnp.concatenate([embp[:, :DH], embp[:, DH:]], axis=0)
    zeros = jnp.zeros((WB, DH), jnp.float32)

    x1_tab, x2_tab = _propagate(tab0, rows, cols, zeros)

    zt = _mean3(tab0.reshape(2 * NP * DH // 128, 128),
                x1_tab.reshape(2 * NP * DH // 128, 128),
                x2_tab.reshape(2 * NP * DH // 128, 128)).reshape(2 * NP, DH)
    z = jnp.concatenate([zt[:NODES], zt[NP:NP + NODES]], axis=1)
    return z[:NUSERS], z[NUSERS:NODES]
